# Initial kernel scaffold; baseline (speedup 1.0000x reference)
import jax
import jax.numpy as jnp
from jax.experimental import pallas as pl


def kernel(x, Wr1, br1, Wr2, br2, We1, be1, We2, be2):
    raise NotImplementedError("write your pallas kernel here")


# speedup vs baseline: 1.1775x; 1.1775x over previous
import functools
from typing import Any

import jax
import jax.numpy as jnp
from jax.experimental import pallas as pl
from jax.experimental.pallas import tpu as pltpu

B = 4096
IN_DIM = 1024
NUM_CLASSES = 1024
NUM_EXPERTS = 8
TOP_K = 2
ROUTER_HIDDEN = 256
EXPERT_HIDDEN = 4096

TM = 256
PPAD = B * TOP_K + NUM_EXPERTS * TM
NMT = PPAD // TM
CHUNK = 512


def _router_body(x_ref, wr1_ref, br1_ref, wr2_ref, br2_ref,
                 probs_ref, pos0_ref, pos1_ref, v0_ref, v1_ref,
                 tile_e_ref, loss_ref):
    x = x_ref[...]
    h = jnp.maximum(jnp.dot(x, wr1_ref[...],
                            preferred_element_type=jnp.float32)
                    + br1_ref[...], 0.0)
    s = jnp.dot(h, wr2_ref[...], preferred_element_type=jnp.float32) \
        + br2_ref[...]
    m = jnp.max(s, axis=1, keepdims=True)
    p = jnp.exp(s - m)
    p = p / jnp.sum(p, axis=1, keepdims=True)
    probs_ref[...] = p

    loss_vec = jnp.sum(p, axis=0, keepdims=True) * (1.0 / B)
    loss_ref[...] = jnp.sum(loss_vec * loss_vec).reshape(1, 1)

    eidx = jax.lax.broadcasted_iota(jnp.int32, (B, NUM_EXPERTS), 1)
    m0 = jnp.max(p, axis=1, keepdims=True)
    e0 = jnp.min(jnp.where(p == m0, eidx, NUM_EXPERTS), axis=1, keepdims=True)
    oh0 = (eidx == e0).astype(jnp.float32)
    pm = jnp.where(oh0 > 0, -jnp.inf, p)
    m1 = jnp.max(pm, axis=1, keepdims=True)
    e1 = jnp.min(jnp.where(pm == m1, eidx, NUM_EXPERTS), axis=1, keepdims=True)
    oh1 = (eidx == e1).astype(jnp.float32)
    v0_ref[...] = m0
    v1_ref[...] = m1

    mm = oh0 + oh1
    cnt = jnp.sum(mm, axis=0, keepdims=True)
    cnti = cnt.astype(jnp.int32)
    padded = ((cnti + (TM - 1)) // TM) * TM
    ltri8 = (jax.lax.broadcasted_iota(jnp.int32, (NUM_EXPERTS, NUM_EXPERTS), 0)
             < jax.lax.broadcasted_iota(jnp.int32, (NUM_EXPERTS, NUM_EXPERTS), 1)
             ).astype(jnp.float32)
    off = jnp.dot(padded.astype(jnp.float32), ltri8,
                  preferred_element_type=jnp.float32)

    tstart = jax.lax.broadcasted_iota(jnp.int32, (NMT, NUM_EXPERTS), 0) * TM
    grp_end = (off + padded.astype(jnp.float32)).astype(jnp.int32)
    tile_e = jnp.sum((tstart >= grp_end).astype(jnp.int32), axis=1,
                     keepdims=True)
    tile_e_ref[...] = jnp.minimum(tile_e, NUM_EXPERTS - 1)

    ltri = (jax.lax.broadcasted_iota(jnp.int32, (CHUNK, CHUNK), 1)
            < jax.lax.broadcasted_iota(jnp.int32, (CHUNK, CHUNK), 0)
            ).astype(jnp.float32)
    run = jnp.zeros((1, NUM_EXPERTS), jnp.float32)
    for c in range(B // CHUNK):
        sl = slice(c * CHUNK, (c + 1) * CHUNK)
        mm_c = mm[sl]
        t = jnp.dot(ltri, mm_c, preferred_element_type=jnp.float32)
        base = off + run + t
        p0 = jnp.sum(oh0[sl] * base, axis=1, keepdims=True)
        p1 = jnp.sum(oh1[sl] * (base + oh0[sl]), axis=1, keepdims=True)
        pos0_ref[sl, :] = p0.astype(jnp.int32)
        pos1_ref[sl, :] = p1.astype(jnp.int32)
        run = run + jnp.sum(mm_c, axis=0, keepdims=True)


@functools.partial(jax.jit, static_argnames=("interpret",))
def _router(x, wr1, br1, wr2, br2, interpret=False):
    outs = pl.pallas_call(
        _router_body,
        out_shape=(
            jax.ShapeDtypeStruct((B, NUM_EXPERTS), jnp.float32),
            jax.ShapeDtypeStruct((B, 1), jnp.int32),
            jax.ShapeDtypeStruct((B, 1), jnp.int32),
            jax.ShapeDtypeStruct((B, 1), jnp.float32),
            jax.ShapeDtypeStruct((B, 1), jnp.float32),
            jax.ShapeDtypeStruct((NMT, 1), jnp.int32),
            jax.ShapeDtypeStruct((1, 1), jnp.float32),
        ),
        interpret=interpret,
    )(x, wr1, br1.reshape(1, ROUTER_HIDDEN), wr2, br2.reshape(1, NUM_EXPERTS))
    return outs


def _gemm1_body(tile_e_ref, xs_ref, w_ref, b_ref, out_ref):
    acc = jnp.dot(xs_ref[...], w_ref[0], preferred_element_type=jnp.float32)
    out_ref[...] = jnp.maximum(acc + b_ref[0], 0.0)


def _gemm2_body(tile_e_ref, h_ref, w_ref, b_ref, out_ref):
    acc = jnp.dot(h_ref[...], w_ref[0], preferred_element_type=jnp.float32)
    out_ref[...] = acc + b_ref[0]


def _grouped_gemm(xs, w, b, tile_e, body, tn, relu_name, interpret=False):
    k = xs.shape[1]
    n = w.shape[2]
    grid = (n // tn, NMT)
    return pl.pallas_call(
        body,
        grid_spec=pltpu.PrefetchScalarGridSpec(
            num_scalar_prefetch=1,
            grid=grid,
            in_specs=[
                pl.BlockSpec((TM, k), lambda ni, mi, te: (mi, 0)),
                pl.BlockSpec((1, k, tn), lambda ni, mi, te: (te[mi], 0, ni)),
                pl.BlockSpec((1, 1, tn), lambda ni, mi, te: (te[mi], 0, ni)),
            ],
            out_specs=pl.BlockSpec((TM, tn), lambda ni, mi, te: (mi, ni)),
        ),
        out_shape=jax.ShapeDtypeStruct((PPAD, n), jnp.float32),
        interpret=interpret,
    )(tile_e, xs, w, b.reshape(NUM_EXPERTS, 1, n))


def kernel(x, Wr1, br1, Wr2, br2, We1, be1, We2, be2):
    probs, pos0, pos1, v0, v1, tile_e, loss = _router(x, Wr1, br1, Wr2, br2)
    pos0 = pos0.reshape(B)
    pos1 = pos1.reshape(B)
    tile_e = tile_e.reshape(NMT)

    xs = jnp.zeros((PPAD, IN_DIM), jnp.float32)
    xs = xs.at[pos0, :].set(x)
    xs = xs.at[pos1, :].set(x)

    h = _grouped_gemm(xs, We1, be1, tile_e, _gemm1_body, 512, "relu")
    y = _grouped_gemm(h, We2, be2, tile_e, _gemm2_body, 512, "none")

    out = (v0 * y[pos0, :] + v1 * y[pos1, :]) * (1.0 / TOP_K)
    return out, probs, loss[0, 0]


# speedup vs baseline: 1.8410x; 1.5635x over previous
import functools
from typing import Any

import jax
import jax.numpy as jnp
from jax.experimental import pallas as pl
from jax.experimental.pallas import tpu as pltpu

B = 4096
IN_DIM = 1024
NUM_CLASSES = 1024
NUM_EXPERTS = 8
TOP_K = 2
ROUTER_HIDDEN = 256
EXPERT_HIDDEN = 4096

TM = 256
PPAD = B * TOP_K + NUM_EXPERTS * TM
NMT = PPAD // TM
CHUNK = 512


def _router_body(x_ref, wr1_ref, br1_ref, wr2_ref, br2_ref,
                 probs_ref, pos0_ref, pos1_ref, v0_ref, v1_ref,
                 tile_e_ref, loss_ref):
    x = x_ref[...]
    h = jnp.maximum(jnp.dot(x, wr1_ref[...],
                            preferred_element_type=jnp.float32)
                    + br1_ref[...], 0.0)
    s = jnp.dot(h, wr2_ref[...], preferred_element_type=jnp.float32) \
        + br2_ref[...]
    m = jnp.max(s, axis=1, keepdims=True)
    p = jnp.exp(s - m)
    p = p / jnp.sum(p, axis=1, keepdims=True)
    probs_ref[...] = p

    loss_vec = jnp.sum(p, axis=0, keepdims=True) * (1.0 / B)
    loss_ref[...] = jnp.sum(loss_vec * loss_vec).reshape(1, 1)

    eidx = jax.lax.broadcasted_iota(jnp.int32, (B, NUM_EXPERTS), 1)
    m0 = jnp.max(p, axis=1, keepdims=True)
    e0 = jnp.min(jnp.where(p == m0, eidx, NUM_EXPERTS), axis=1, keepdims=True)
    oh0 = (eidx == e0).astype(jnp.float32)
    pm = jnp.where(oh0 > 0, -jnp.inf, p)
    m1 = jnp.max(pm, axis=1, keepdims=True)
    e1 = jnp.min(jnp.where(pm == m1, eidx, NUM_EXPERTS), axis=1, keepdims=True)
    oh1 = (eidx == e1).astype(jnp.float32)
    v0_ref[...] = m0
    v1_ref[...] = m1

    mm = oh0 + oh1
    cnt = jnp.sum(mm, axis=0, keepdims=True)
    cnti = cnt.astype(jnp.int32)
    padded = ((cnti + (TM - 1)) // TM) * TM
    ltri8 = (jax.lax.broadcasted_iota(jnp.int32, (NUM_EXPERTS, NUM_EXPERTS), 0)
             < jax.lax.broadcasted_iota(jnp.int32, (NUM_EXPERTS, NUM_EXPERTS), 1)
             ).astype(jnp.float32)
    off = jnp.dot(padded.astype(jnp.float32), ltri8,
                  preferred_element_type=jnp.float32)

    tstart = jax.lax.broadcasted_iota(jnp.int32, (NMT, NUM_EXPERTS), 0) * TM
    grp_end = (off + padded.astype(jnp.float32)).astype(jnp.int32)
    tile_e = jnp.sum((tstart >= grp_end).astype(jnp.int32), axis=1,
                     keepdims=True)
    tile_e_ref[...] = jnp.minimum(tile_e, NUM_EXPERTS - 1)

    ltri = (jax.lax.broadcasted_iota(jnp.int32, (CHUNK, CHUNK), 1)
            < jax.lax.broadcasted_iota(jnp.int32, (CHUNK, CHUNK), 0)
            ).astype(jnp.float32)
    run = jnp.zeros((1, NUM_EXPERTS), jnp.float32)
    for c in range(B // CHUNK):
        sl = slice(c * CHUNK, (c + 1) * CHUNK)
        mm_c = mm[sl]
        t = jnp.dot(ltri, mm_c, preferred_element_type=jnp.float32)
        base = off + run + t
        p0 = jnp.sum(oh0[sl] * base, axis=1, keepdims=True)
        p1 = jnp.sum(oh1[sl] * (base + oh0[sl]), axis=1, keepdims=True)
        pos0_ref[sl, :] = p0.astype(jnp.int32)
        pos1_ref[sl, :] = p1.astype(jnp.int32)
        run = run + jnp.sum(mm_c, axis=0, keepdims=True)


@functools.partial(jax.jit, static_argnames=("interpret",))
def _router(x, wr1, br1, wr2, br2, interpret=False):
    outs = pl.pallas_call(
        _router_body,
        out_shape=(
            jax.ShapeDtypeStruct((B, NUM_EXPERTS), jnp.float32),
            jax.ShapeDtypeStruct((B, 1), jnp.int32),
            jax.ShapeDtypeStruct((B, 1), jnp.int32),
            jax.ShapeDtypeStruct((B, 1), jnp.float32),
            jax.ShapeDtypeStruct((B, 1), jnp.float32),
            jax.ShapeDtypeStruct((NMT, 1), jnp.int32),
            jax.ShapeDtypeStruct((1, 1), jnp.float32),
        ),
        interpret=interpret,
    )(x, wr1, br1.reshape(1, ROUTER_HIDDEN), wr2, br2.reshape(1, NUM_EXPERTS))
    return outs


def _gemm1_body(tile_e_ref, xs_ref, w_ref, b_ref, out_ref):
    acc = jnp.dot(xs_ref[...], w_ref[0], preferred_element_type=jnp.float32)
    out_ref[...] = jnp.maximum(acc + b_ref[0], 0.0)


def _gemm2_body(tile_e_ref, h_ref, w_ref, b_ref, out_ref):
    acc = jnp.dot(h_ref[...], w_ref[0], preferred_element_type=jnp.float32)
    out_ref[...] = acc + b_ref[0]


def _grouped_gemm(xs, w, b, tile_e, body, tn, relu_name, interpret=False):
    k = xs.shape[1]
    n = w.shape[2]
    grid = (n // tn, NMT)
    return pl.pallas_call(
        body,
        grid_spec=pltpu.PrefetchScalarGridSpec(
            num_scalar_prefetch=1,
            grid=grid,
            in_specs=[
                pl.BlockSpec((TM, k), lambda ni, mi, te: (mi, 0)),
                pl.BlockSpec((1, k, tn), lambda ni, mi, te: (te[mi], 0, ni)),
                pl.BlockSpec((1, 1, tn), lambda ni, mi, te: (te[mi], 0, ni)),
            ],
            out_specs=pl.BlockSpec((TM, tn), lambda ni, mi, te: (mi, ni)),
        ),
        out_shape=jax.ShapeDtypeStruct((PPAD, n), jnp.float32),
        interpret=interpret,
    )(tile_e, xs, w, b.reshape(NUM_EXPERTS, 1, n))


def kernel(x, Wr1, br1, Wr2, br2, We1, be1, We2, be2):
    probs, pos0, pos1, v0, v1, tile_e, loss = _router(x, Wr1, br1, Wr2, br2)
    pos0 = pos0.reshape(B)
    pos1 = pos1.reshape(B)
    tile_e = tile_e.reshape(NMT)

    xs = jnp.zeros((PPAD, IN_DIM), jnp.float32)
    xs = xs.at[pos0, :].set(x)
    xs = xs.at[pos1, :].set(x)

    h = _grouped_gemm(xs, We1, be1, tile_e, _gemm1_body, EXPERT_HIDDEN, "relu")
    y = _grouped_gemm(h, We2, be2, tile_e, _gemm2_body, NUM_CLASSES, "none")

    out = (v0 * y[pos0, :] + v1 * y[pos1, :]) * (1.0 / TOP_K)
    return out, probs, loss[0, 0]


# speedup vs baseline: 2.0049x; 1.0890x over previous
import functools
from typing import Any

import jax
import jax.numpy as jnp
from jax import lax
from jax.experimental import pallas as pl
from jax.experimental.pallas import tpu as pltpu
from jax.experimental.pallas import tpu_sc as plsc

B = 4096
IN_DIM = 1024
NUM_CLASSES = 1024
NUM_EXPERTS = 8
TOP_K = 2
ROUTER_HIDDEN = 256
EXPERT_HIDDEN = 4096

TM = 256
PPAD = B * TOP_K + NUM_EXPERTS * TM
NMT = PPAD // TM
CHUNK = 512

SC_CORES = 2
SC_SUBCORES = 16
NW = SC_CORES * SC_SUBCORES
TOK_W = B // NW
CCH = 16
NCH = TOK_W // CCH
LANES = 16


def _router_body(x_ref, wr1_ref, br1_ref, wr2_ref, br2_ref,
                 probs_ref, pos0_ref, pos1_ref, v0_ref, v1_ref,
                 tile_e_ref, loss_ref):
    x = x_ref[...]
    h = jnp.maximum(jnp.dot(x, wr1_ref[...],
                            preferred_element_type=jnp.float32)
                    + br1_ref[...], 0.0)
    s = jnp.dot(h, wr2_ref[...], preferred_element_type=jnp.float32) \
        + br2_ref[...]
    m = jnp.max(s, axis=1, keepdims=True)
    p = jnp.exp(s - m)
    p = p / jnp.sum(p, axis=1, keepdims=True)
    probs_ref[...] = p

    loss_vec = jnp.sum(p, axis=0, keepdims=True) * (1.0 / B)
    loss_ref[...] = jnp.sum(loss_vec * loss_vec).reshape(1, 1)

    eidx = jax.lax.broadcasted_iota(jnp.int32, (B, NUM_EXPERTS), 1)
    m0 = jnp.max(p, axis=1, keepdims=True)
    e0 = jnp.min(jnp.where(p == m0, eidx, NUM_EXPERTS), axis=1, keepdims=True)
    oh0 = (eidx == e0).astype(jnp.float32)
    pm = jnp.where(oh0 > 0, -jnp.inf, p)
    m1 = jnp.max(pm, axis=1, keepdims=True)
    e1 = jnp.min(jnp.where(pm == m1, eidx, NUM_EXPERTS), axis=1, keepdims=True)
    oh1 = (eidx == e1).astype(jnp.float32)
    v0_ref[...] = m0
    v1_ref[...] = m1

    mm = oh0 + oh1
    cnt = jnp.sum(mm, axis=0, keepdims=True)
    cnti = cnt.astype(jnp.int32)
    padded = ((cnti + (TM - 1)) // TM) * TM
    ltri8 = (jax.lax.broadcasted_iota(jnp.int32, (NUM_EXPERTS, NUM_EXPERTS), 0)
             < jax.lax.broadcasted_iota(jnp.int32, (NUM_EXPERTS, NUM_EXPERTS), 1)
             ).astype(jnp.float32)
    off = jnp.dot(padded.astype(jnp.float32), ltri8,
                  preferred_element_type=jnp.float32)

    tstart = jax.lax.broadcasted_iota(jnp.int32, (NMT, NUM_EXPERTS), 0) * TM
    grp_end = (off + padded.astype(jnp.float32)).astype(jnp.int32)
    tile_e = jnp.sum((tstart >= grp_end).astype(jnp.int32), axis=1,
                     keepdims=True)
    tile_e_ref[...] = jnp.minimum(tile_e, NUM_EXPERTS - 1)

    ltri = (jax.lax.broadcasted_iota(jnp.int32, (CHUNK, CHUNK), 1)
            < jax.lax.broadcasted_iota(jnp.int32, (CHUNK, CHUNK), 0)
            ).astype(jnp.float32)
    run = jnp.zeros((1, NUM_EXPERTS), jnp.float32)
    for c in range(B // CHUNK):
        sl = slice(c * CHUNK, (c + 1) * CHUNK)
        mm_c = mm[sl]
        t = jnp.dot(ltri, mm_c, preferred_element_type=jnp.float32)
        base = off + run + t
        p0 = jnp.sum(oh0[sl] * base, axis=1, keepdims=True)
        p1 = jnp.sum(oh1[sl] * (base + oh0[sl]), axis=1, keepdims=True)
        pos0_ref[sl, :] = p0.astype(jnp.int32)
        pos1_ref[sl, :] = p1.astype(jnp.int32)
        run = run + jnp.sum(mm_c, axis=0, keepdims=True)


@functools.partial(jax.jit, static_argnames=("interpret",))
def _router(x, wr1, br1, wr2, br2, interpret=False):
    outs = pl.pallas_call(
        _router_body,
        out_shape=(
            jax.ShapeDtypeStruct((B, NUM_EXPERTS), jnp.float32),
            jax.ShapeDtypeStruct((B, 1), jnp.int32),
            jax.ShapeDtypeStruct((B, 1), jnp.int32),
            jax.ShapeDtypeStruct((B, 1), jnp.float32),
            jax.ShapeDtypeStruct((B, 1), jnp.float32),
            jax.ShapeDtypeStruct((NMT, 1), jnp.int32),
            jax.ShapeDtypeStruct((1, 1), jnp.float32),
        ),
        interpret=interpret,
    )(x, wr1, br1.reshape(1, ROUTER_HIDDEN), wr2, br2.reshape(1, NUM_EXPERTS))
    return outs


def _gemm1_body(tile_e_ref, xs_ref, w_ref, b_ref, out_ref):
    acc = jnp.dot(xs_ref[...], w_ref[0], preferred_element_type=jnp.float32)
    out_ref[...] = jnp.maximum(acc + b_ref[0], 0.0)


def _gemm2_body(tile_e_ref, h_ref, w_ref, b_ref, out_ref):
    acc = jnp.dot(h_ref[...], w_ref[0], preferred_element_type=jnp.float32)
    out_ref[...] = acc + b_ref[0]


def _grouped_gemm(xs, w, b, tile_e, body, tn, relu_name, interpret=False):
    k = xs.shape[1]
    n = w.shape[2]
    grid = (n // tn, NMT)
    return pl.pallas_call(
        body,
        grid_spec=pltpu.PrefetchScalarGridSpec(
            num_scalar_prefetch=1,
            grid=grid,
            in_specs=[
                pl.BlockSpec((TM, k), lambda ni, mi, te: (mi, 0)),
                pl.BlockSpec((1, k, tn), lambda ni, mi, te: (te[mi], 0, ni)),
                pl.BlockSpec((1, 1, tn), lambda ni, mi, te: (te[mi], 0, ni)),
            ],
            out_specs=pl.BlockSpec((TM, tn), lambda ni, mi, te: (mi, ni)),
        ),
        out_shape=jax.ShapeDtypeStruct((PPAD, n), jnp.float32),
        interpret=interpret,
    )(tile_e, xs, w, b.reshape(NUM_EXPERTS, 1, n))


def _dispatch_body(x_hbm, p0_hbm, p1_hbm, xs_hbm, xbuf, d0, d1, sem):
    wid = lax.axis_index("s") * SC_CORES + lax.axis_index("c")
    base = wid * TOK_W

    def chunk(k, carry):
        tb = pl.multiple_of(base + k * CCH, CCH)
        pltpu.sync_copy(p0_hbm.at[pl.ds(tb, CCH)], d0)
        pltpu.sync_copy(p1_hbm.at[pl.ds(tb, CCH)], d1)
        pltpu.sync_copy(x_hbm.at[pl.ds(tb, CCH)], xbuf)
        c0 = pltpu.make_async_copy(xbuf, xs_hbm.at[d0], sem)
        c1 = pltpu.make_async_copy(xbuf, xs_hbm.at[d1], sem)
        c0.start()
        c1.start()
        c0.wait()
        c1.wait()
        return carry

    lax.fori_loop(0, NCH, chunk, 0)


def _combine_body(y_hbm, p0_hbm, p1_hbm, v0_hbm, v1_hbm, out_hbm,
                  b0, b1, ob, d0, d1, w0, w1, sem):
    wid = lax.axis_index("s") * SC_CORES + lax.axis_index("c")
    base = wid * TOK_W

    def chunk(k, carry):
        tb = pl.multiple_of(base + k * CCH, CCH)
        pltpu.sync_copy(p0_hbm.at[pl.ds(tb, CCH)], d0)
        pltpu.sync_copy(p1_hbm.at[pl.ds(tb, CCH)], d1)
        pltpu.sync_copy(v0_hbm.at[pl.ds(tb, CCH)], w0)
        pltpu.sync_copy(v1_hbm.at[pl.ds(tb, CCH)], w1)
        g0 = pltpu.make_async_copy(y_hbm.at[d0], b0, sem)
        g1 = pltpu.make_async_copy(y_hbm.at[d1], b1, sem)
        g0.start()
        g1.start()
        g0.wait()
        g1.wait()

        ww0 = w0[...] * 0.5
        ww1 = w1[...] * 0.5
        gdn = lax.GatherDimensionNumbers(offset_dims=(),
                                         collapsed_slice_dims=(0,),
                                         start_index_map=(0,))

        def row(r, rcarry):
            ridx = jnp.zeros((LANES, 1), jnp.int32) + r
            wv0 = lax.gather(ww0, ridx, gdn, (1,),
                             mode=lax.GatherScatterMode.PROMISE_IN_BOUNDS)
            wv1 = lax.gather(ww1, ridx, gdn, (1,),
                             mode=lax.GatherScatterMode.PROMISE_IN_BOUNDS)

            def col(c, ccarry):
                sl = pl.ds(c * LANES, LANES)
                ob[r, sl] = b0[r, sl] * wv0 + b1[r, sl] * wv1
                return ccarry

            lax.fori_loop(0, NUM_CLASSES // LANES, col, 0)
            return rcarry

        lax.fori_loop(0, CCH, row, 0)
        pltpu.sync_copy(ob, out_hbm.at[pl.ds(tb, CCH)])
        return carry

    lax.fori_loop(0, NCH, chunk, 0)


_SC_MESH = plsc.VectorSubcoreMesh(core_axis_name="c", subcore_axis_name="s")


def _sc_dispatch(x, pos0, pos1):
    return pl.kernel(
        _dispatch_body,
        out_type=jax.ShapeDtypeStruct((PPAD, IN_DIM), jnp.float32),
        mesh=_SC_MESH,
        scratch_types=[
            pltpu.VMEM((CCH, IN_DIM), jnp.float32),
            pltpu.VMEM((CCH,), jnp.int32),
            pltpu.VMEM((CCH,), jnp.int32),
            pltpu.SemaphoreType.DMA,
        ],
    )(x, pos0, pos1)


def _sc_combine(y, pos0, pos1, v0, v1):
    return pl.kernel(
        _combine_body,
        out_type=jax.ShapeDtypeStruct((B, NUM_CLASSES), jnp.float32),
        mesh=_SC_MESH,
        scratch_types=[
            pltpu.VMEM((CCH, NUM_CLASSES), jnp.float32),
            pltpu.VMEM((CCH, NUM_CLASSES), jnp.float32),
            pltpu.VMEM((CCH, NUM_CLASSES), jnp.float32),
            pltpu.VMEM((CCH,), jnp.int32),
            pltpu.VMEM((CCH,), jnp.int32),
            pltpu.VMEM((CCH,), jnp.float32),
            pltpu.VMEM((CCH,), jnp.float32),
            pltpu.SemaphoreType.DMA,
        ],
    )(y, pos0, pos1, v0, v1)


def kernel(x, Wr1, br1, Wr2, br2, We1, be1, We2, be2):
    probs, pos0, pos1, v0, v1, tile_e, loss = _router(x, Wr1, br1, Wr2, br2)
    pos0 = pos0.reshape(B)
    pos1 = pos1.reshape(B)
    tile_e = tile_e.reshape(NMT)

    xs = _sc_dispatch(x, pos0, pos1)

    h = _grouped_gemm(xs, We1, be1, tile_e, _gemm1_body, EXPERT_HIDDEN, "relu")
    y = _grouped_gemm(h, We2, be2, tile_e, _gemm2_body, NUM_CLASSES, "none")

    out = _sc_combine(y, pos0, pos1, v0.reshape(B), v1.reshape(B))
    return out, probs, loss[0, 0]


# speedup vs baseline: 2.0732x; 1.0340x over previous
import functools
from typing import Any

import jax
import jax.numpy as jnp
from jax import lax
from jax.experimental import pallas as pl
from jax.experimental.pallas import tpu as pltpu
from jax.experimental.pallas import tpu_sc as plsc

B = 4096
IN_DIM = 1024
NUM_CLASSES = 1024
NUM_EXPERTS = 8
TOP_K = 2
ROUTER_HIDDEN = 256
EXPERT_HIDDEN = 4096

TM = 256
PPAD = B * TOP_K + NUM_EXPERTS * TM
NMT = PPAD // TM
CHUNK = 512

SC_CORES = 2
SC_SUBCORES = 16
NW = SC_CORES * SC_SUBCORES
TOK_W = B // NW
CCH = 16
NCH = TOK_W // CCH
LANES = 16


def _router_body(x_ref, wr1_ref, br1_ref, wr2_ref, br2_ref,
                 probs_ref, pos0_ref, pos1_ref, v0_ref, v1_ref,
                 tile_e_ref, loss_ref):
    x = x_ref[...]
    h = jnp.maximum(jnp.dot(x, wr1_ref[...],
                            preferred_element_type=jnp.float32)
                    + br1_ref[...], 0.0)
    s = jnp.dot(h, wr2_ref[...], preferred_element_type=jnp.float32) \
        + br2_ref[...]
    m = jnp.max(s, axis=1, keepdims=True)
    p = jnp.exp(s - m)
    p = p / jnp.sum(p, axis=1, keepdims=True)
    probs_ref[...] = p

    loss_vec = jnp.sum(p, axis=0, keepdims=True) * (1.0 / B)
    loss_ref[...] = jnp.sum(loss_vec * loss_vec).reshape(1, 1)

    eidx = jax.lax.broadcasted_iota(jnp.int32, (B, NUM_EXPERTS), 1)
    m0 = jnp.max(p, axis=1, keepdims=True)
    e0 = jnp.min(jnp.where(p == m0, eidx, NUM_EXPERTS), axis=1, keepdims=True)
    oh0 = (eidx == e0).astype(jnp.float32)
    pm = jnp.where(oh0 > 0, -jnp.inf, p)
    m1 = jnp.max(pm, axis=1, keepdims=True)
    e1 = jnp.min(jnp.where(pm == m1, eidx, NUM_EXPERTS), axis=1, keepdims=True)
    oh1 = (eidx == e1).astype(jnp.float32)
    v0_ref[...] = m0
    v1_ref[...] = m1

    mm = oh0 + oh1
    cnt = jnp.sum(mm, axis=0, keepdims=True)
    cnti = cnt.astype(jnp.int32)
    padded = ((cnti + (TM - 1)) // TM) * TM
    ltri8 = (jax.lax.broadcasted_iota(jnp.int32, (NUM_EXPERTS, NUM_EXPERTS), 0)
             < jax.lax.broadcasted_iota(jnp.int32, (NUM_EXPERTS, NUM_EXPERTS), 1)
             ).astype(jnp.float32)
    off = jnp.dot(padded.astype(jnp.float32), ltri8,
                  preferred_element_type=jnp.float32)

    tstart = jax.lax.broadcasted_iota(jnp.int32, (NMT, NUM_EXPERTS), 0) * TM
    grp_end = (off + padded.astype(jnp.float32)).astype(jnp.int32)
    tile_e = jnp.sum((tstart >= grp_end).astype(jnp.int32), axis=1,
                     keepdims=True)
    tile_e_ref[...] = jnp.minimum(tile_e, NUM_EXPERTS - 1)

    ltri = (jax.lax.broadcasted_iota(jnp.int32, (CHUNK, CHUNK), 1)
            < jax.lax.broadcasted_iota(jnp.int32, (CHUNK, CHUNK), 0)
            ).astype(jnp.float32)
    run = jnp.zeros((1, NUM_EXPERTS), jnp.float32)
    for c in range(B // CHUNK):
        sl = slice(c * CHUNK, (c + 1) * CHUNK)
        mm_c = mm[sl]
        t = jnp.dot(ltri, mm_c, preferred_element_type=jnp.float32)
        base = off + run + t
        p0 = jnp.sum(oh0[sl] * base, axis=1, keepdims=True)
        p1 = jnp.sum(oh1[sl] * (base + oh0[sl]), axis=1, keepdims=True)
        pos0_ref[sl, :] = p0.astype(jnp.int32)
        pos1_ref[sl, :] = p1.astype(jnp.int32)
        run = run + jnp.sum(mm_c, axis=0, keepdims=True)


@functools.partial(jax.jit, static_argnames=("interpret",))
def _router(x, wr1, br1, wr2, br2, interpret=False):
    outs = pl.pallas_call(
        _router_body,
        out_shape=(
            jax.ShapeDtypeStruct((B, NUM_EXPERTS), jnp.float32),
            jax.ShapeDtypeStruct((B, 1), jnp.int32),
            jax.ShapeDtypeStruct((B, 1), jnp.int32),
            jax.ShapeDtypeStruct((B, 1), jnp.float32),
            jax.ShapeDtypeStruct((B, 1), jnp.float32),
            jax.ShapeDtypeStruct((NMT, 1), jnp.int32),
            jax.ShapeDtypeStruct((1, 1), jnp.float32),
        ),
        interpret=interpret,
    )(x, wr1, br1.reshape(1, ROUTER_HIDDEN), wr2, br2.reshape(1, NUM_EXPERTS))
    return outs


def _gemm1_body(tile_e_ref, xs_ref, w_ref, b_ref, out_ref):
    acc = jnp.dot(xs_ref[...].astype(jnp.bfloat16),
                  w_ref[0].astype(jnp.bfloat16),
                  preferred_element_type=jnp.float32)
    out_ref[...] = jnp.maximum(acc + b_ref[0], 0.0).astype(jnp.bfloat16)


def _gemm2_body(tile_e_ref, h_ref, w_ref, b_ref, out_ref):
    acc = jnp.dot(h_ref[...], w_ref[0].astype(jnp.bfloat16),
                  preferred_element_type=jnp.float32)
    out_ref[...] = acc + b_ref[0]


def _grouped_gemm(xs, w, b, tile_e, body, tn, out_dtype, interpret=False):
    k = xs.shape[1]
    n = w.shape[2]
    grid = (n // tn, NMT)
    return pl.pallas_call(
        body,
        grid_spec=pltpu.PrefetchScalarGridSpec(
            num_scalar_prefetch=1,
            grid=grid,
            in_specs=[
                pl.BlockSpec((TM, k), lambda ni, mi, te: (mi, 0)),
                pl.BlockSpec((1, k, tn), lambda ni, mi, te: (te[mi], 0, ni)),
                pl.BlockSpec((1, 1, tn), lambda ni, mi, te: (te[mi], 0, ni)),
            ],
            out_specs=pl.BlockSpec((TM, tn), lambda ni, mi, te: (mi, ni)),
        ),
        out_shape=jax.ShapeDtypeStruct((PPAD, n), out_dtype),
        interpret=interpret,
    )(tile_e, xs, w, b.reshape(NUM_EXPERTS, 1, n))


def _dispatch_body(x_hbm, p0_hbm, p1_hbm, xs_hbm, xbuf, d0, d1, sem):
    wid = lax.axis_index("s") * SC_CORES + lax.axis_index("c")
    base = wid * TOK_W

    def chunk(k, carry):
        tb = pl.multiple_of(base + k * CCH, CCH)
        pltpu.sync_copy(p0_hbm.at[pl.ds(tb, CCH)], d0)
        pltpu.sync_copy(p1_hbm.at[pl.ds(tb, CCH)], d1)
        pltpu.sync_copy(x_hbm.at[pl.ds(tb, CCH)], xbuf)
        c0 = pltpu.make_async_copy(xbuf, xs_hbm.at[d0], sem)
        c1 = pltpu.make_async_copy(xbuf, xs_hbm.at[d1], sem)
        c0.start()
        c1.start()
        c0.wait()
        c1.wait()
        return carry

    lax.fori_loop(0, NCH, chunk, 0)


def _combine_body(y_hbm, p0_hbm, p1_hbm, v0_hbm, v1_hbm, out_hbm,
                  b0, b1, ob, d0, d1, w0, w1, sem):
    wid = lax.axis_index("s") * SC_CORES + lax.axis_index("c")
    base = wid * TOK_W

    def chunk(k, carry):
        tb = pl.multiple_of(base + k * CCH, CCH)
        pltpu.sync_copy(p0_hbm.at[pl.ds(tb, CCH)], d0)
        pltpu.sync_copy(p1_hbm.at[pl.ds(tb, CCH)], d1)
        pltpu.sync_copy(v0_hbm.at[pl.ds(tb, CCH)], w0)
        pltpu.sync_copy(v1_hbm.at[pl.ds(tb, CCH)], w1)
        g0 = pltpu.make_async_copy(y_hbm.at[d0], b0, sem)
        g1 = pltpu.make_async_copy(y_hbm.at[d1], b1, sem)
        g0.start()
        g1.start()
        g0.wait()
        g1.wait()

        ww0 = w0[...] * 0.5
        ww1 = w1[...] * 0.5
        gdn = lax.GatherDimensionNumbers(offset_dims=(),
                                         collapsed_slice_dims=(0,),
                                         start_index_map=(0,))

        def row(r, rcarry):
            ridx = jnp.zeros((LANES, 1), jnp.int32) + r
            wv0 = lax.gather(ww0, ridx, gdn, (1,),
                             mode=lax.GatherScatterMode.PROMISE_IN_BOUNDS)
            wv1 = lax.gather(ww1, ridx, gdn, (1,),
                             mode=lax.GatherScatterMode.PROMISE_IN_BOUNDS)

            def col(c, ccarry):
                sl = pl.ds(c * LANES, LANES)
                ob[r, sl] = b0[r, sl] * wv0 + b1[r, sl] * wv1
                return ccarry

            lax.fori_loop(0, NUM_CLASSES // LANES, col, 0)
            return rcarry

        lax.fori_loop(0, CCH, row, 0)
        pltpu.sync_copy(ob, out_hbm.at[pl.ds(tb, CCH)])
        return carry

    lax.fori_loop(0, NCH, chunk, 0)


_SC_MESH = plsc.VectorSubcoreMesh(core_axis_name="c", subcore_axis_name="s")


def _sc_dispatch(x, pos0, pos1):
    return pl.kernel(
        _dispatch_body,
        out_type=jax.ShapeDtypeStruct((PPAD, IN_DIM), jnp.float32),
        mesh=_SC_MESH,
        scratch_types=[
            pltpu.VMEM((CCH, IN_DIM), jnp.float32),
            pltpu.VMEM((CCH,), jnp.int32),
            pltpu.VMEM((CCH,), jnp.int32),
            pltpu.SemaphoreType.DMA,
        ],
    )(x, pos0, pos1)


def _sc_combine(y, pos0, pos1, v0, v1):
    return pl.kernel(
        _combine_body,
        out_type=jax.ShapeDtypeStruct((B, NUM_CLASSES), jnp.float32),
        mesh=_SC_MESH,
        scratch_types=[
            pltpu.VMEM((CCH, NUM_CLASSES), jnp.float32),
            pltpu.VMEM((CCH, NUM_CLASSES), jnp.float32),
            pltpu.VMEM((CCH, NUM_CLASSES), jnp.float32),
            pltpu.VMEM((CCH,), jnp.int32),
            pltpu.VMEM((CCH,), jnp.int32),
            pltpu.VMEM((CCH,), jnp.float32),
            pltpu.VMEM((CCH,), jnp.float32),
            pltpu.SemaphoreType.DMA,
        ],
    )(y, pos0, pos1, v0, v1)


def kernel(x, Wr1, br1, Wr2, br2, We1, be1, We2, be2):
    probs, pos0, pos1, v0, v1, tile_e, loss = _router(x, Wr1, br1, Wr2, br2)
    pos0 = pos0.reshape(B)
    pos1 = pos1.reshape(B)
    tile_e = tile_e.reshape(NMT)

    xs = _sc_dispatch(x, pos0, pos1)

    h = _grouped_gemm(xs, We1, be1, tile_e, _gemm1_body, EXPERT_HIDDEN,
                      jnp.bfloat16)
    y = _grouped_gemm(h, We2, be2, tile_e, _gemm2_body, NUM_CLASSES,
                      jnp.float32)

    out = _sc_combine(y, pos0, pos1, v0.reshape(B), v1.reshape(B))
    return out, probs, loss[0, 0]


# speedup vs baseline: 2.0754x; 1.0011x over previous
import functools
from typing import Any

import jax
import jax.numpy as jnp
from jax import lax
from jax.experimental import pallas as pl
from jax.experimental.pallas import tpu as pltpu
from jax.experimental.pallas import tpu_sc as plsc

B = 4096
IN_DIM = 1024
NUM_CLASSES = 1024
NUM_EXPERTS = 8
TOP_K = 2
ROUTER_HIDDEN = 256
EXPERT_HIDDEN = 4096

TM = 256
PPAD = B * TOP_K + NUM_EXPERTS * TM
NMT = PPAD // TM
CHUNK = 512

SC_CORES = 2
SC_SUBCORES = 16
NW = SC_CORES * SC_SUBCORES
TOK_W = B // NW
CCH = 16
NCH = TOK_W // CCH
LANES = 16


def _router_body(x_ref, wr1_ref, br1_ref, wr2_ref, br2_ref,
                 probs_ref, pos0_ref, pos1_ref, v0_ref, v1_ref,
                 tile_e_ref, loss_ref):
    x = x_ref[...]
    h = jnp.maximum(jnp.dot(x, wr1_ref[...],
                            preferred_element_type=jnp.float32)
                    + br1_ref[...], 0.0)
    s = jnp.dot(h, wr2_ref[...], preferred_element_type=jnp.float32) \
        + br2_ref[...]
    m = jnp.max(s, axis=1, keepdims=True)
    p = jnp.exp(s - m)
    p = p / jnp.sum(p, axis=1, keepdims=True)
    probs_ref[...] = p

    loss_vec = jnp.sum(p, axis=0, keepdims=True) * (1.0 / B)
    loss_ref[...] = jnp.sum(loss_vec * loss_vec).reshape(1, 1)

    eidx = jax.lax.broadcasted_iota(jnp.int32, (B, NUM_EXPERTS), 1)
    m0 = jnp.max(p, axis=1, keepdims=True)
    e0 = jnp.min(jnp.where(p == m0, eidx, NUM_EXPERTS), axis=1, keepdims=True)
    oh0 = (eidx == e0).astype(jnp.float32)
    pm = jnp.where(oh0 > 0, -jnp.inf, p)
    m1 = jnp.max(pm, axis=1, keepdims=True)
    e1 = jnp.min(jnp.where(pm == m1, eidx, NUM_EXPERTS), axis=1, keepdims=True)
    oh1 = (eidx == e1).astype(jnp.float32)
    v0_ref[...] = m0
    v1_ref[...] = m1

    mm = oh0 + oh1
    cnt = jnp.sum(mm, axis=0, keepdims=True)
    cnti = cnt.astype(jnp.int32)
    padded = ((cnti + (TM - 1)) // TM) * TM
    ltri8 = (jax.lax.broadcasted_iota(jnp.int32, (NUM_EXPERTS, NUM_EXPERTS), 0)
             < jax.lax.broadcasted_iota(jnp.int32, (NUM_EXPERTS, NUM_EXPERTS), 1)
             ).astype(jnp.float32)
    off = jnp.dot(padded.astype(jnp.float32), ltri8,
                  preferred_element_type=jnp.float32)

    tstart = jax.lax.broadcasted_iota(jnp.int32, (NMT, NUM_EXPERTS), 0) * TM
    grp_end = (off + padded.astype(jnp.float32)).astype(jnp.int32)
    tile_e = jnp.sum((tstart >= grp_end).astype(jnp.int32), axis=1,
                     keepdims=True)
    tile_e_ref[...] = jnp.minimum(tile_e, NUM_EXPERTS - 1)

    ltri = (jax.lax.broadcasted_iota(jnp.int32, (CHUNK, CHUNK), 1)
            < jax.lax.broadcasted_iota(jnp.int32, (CHUNK, CHUNK), 0)
            ).astype(jnp.float32)
    run = jnp.zeros((1, NUM_EXPERTS), jnp.float32)
    for c in range(B // CHUNK):
        sl = slice(c * CHUNK, (c + 1) * CHUNK)
        mm_c = mm[sl]
        t = jnp.dot(ltri, mm_c, preferred_element_type=jnp.float32)
        base = off + run + t
        p0 = jnp.sum(oh0[sl] * base, axis=1, keepdims=True)
        p1 = jnp.sum(oh1[sl] * (base + oh0[sl]), axis=1, keepdims=True)
        pos0_ref[sl, :] = p0.astype(jnp.int32)
        pos1_ref[sl, :] = p1.astype(jnp.int32)
        run = run + jnp.sum(mm_c, axis=0, keepdims=True)


@functools.partial(jax.jit, static_argnames=("interpret",))
def _router(x, wr1, br1, wr2, br2, interpret=False):
    outs = pl.pallas_call(
        _router_body,
        out_shape=(
            jax.ShapeDtypeStruct((B, NUM_EXPERTS), jnp.float32),
            jax.ShapeDtypeStruct((B, 1), jnp.int32),
            jax.ShapeDtypeStruct((B, 1), jnp.int32),
            jax.ShapeDtypeStruct((B, 1), jnp.float32),
            jax.ShapeDtypeStruct((B, 1), jnp.float32),
            jax.ShapeDtypeStruct((NMT, 1), jnp.int32),
            jax.ShapeDtypeStruct((1, 1), jnp.float32),
        ),
        interpret=interpret,
    )(x, wr1, br1.reshape(1, ROUTER_HIDDEN), wr2, br2.reshape(1, NUM_EXPERTS))
    return outs


def _gemm1_body(tile_e_ref, xs_ref, w_ref, b_ref, out_ref):
    acc = jnp.dot(xs_ref[...].astype(jnp.bfloat16),
                  w_ref[0].astype(jnp.bfloat16),
                  preferred_element_type=jnp.float32)
    out_ref[...] = jnp.maximum(acc + b_ref[0], 0.0).astype(jnp.bfloat16)


def _gemm2_body(tile_e_ref, h_ref, w_ref, b_ref, out_ref):
    acc = jnp.dot(h_ref[...], w_ref[0].astype(jnp.bfloat16),
                  preferred_element_type=jnp.float32)
    out_ref[...] = acc + b_ref[0]


def _grouped_gemm(xs, w, b, tile_e, body, tn, out_dtype, interpret=False):
    k = xs.shape[1]
    n = w.shape[2]
    grid = (n // tn, NMT)
    return pl.pallas_call(
        body,
        grid_spec=pltpu.PrefetchScalarGridSpec(
            num_scalar_prefetch=1,
            grid=grid,
            in_specs=[
                pl.BlockSpec((TM, k), lambda ni, mi, te: (mi, 0)),
                pl.BlockSpec((1, k, tn), lambda ni, mi, te: (te[mi], 0, ni)),
                pl.BlockSpec((1, 1, tn), lambda ni, mi, te: (te[mi], 0, ni)),
            ],
            out_specs=pl.BlockSpec((TM, tn), lambda ni, mi, te: (mi, ni)),
        ),
        out_shape=jax.ShapeDtypeStruct((PPAD, n), out_dtype),
        interpret=interpret,
    )(tile_e, xs, w, b.reshape(NUM_EXPERTS, 1, n))


def _dispatch_body(x_hbm, p0_hbm, p1_hbm, xs_hbm, xbuf, d0, d1, sem):
    wid = lax.axis_index("s") * SC_CORES + lax.axis_index("c")
    base = wid * TOK_W

    def chunk(k, carry):
        tb = pl.multiple_of(base + k * CCH, CCH)
        pltpu.sync_copy(p0_hbm.at[pl.ds(tb, CCH)], d0)
        pltpu.sync_copy(p1_hbm.at[pl.ds(tb, CCH)], d1)
        pltpu.sync_copy(x_hbm.at[pl.ds(tb, CCH)], xbuf)
        c0 = pltpu.make_async_copy(xbuf, xs_hbm.at[d0], sem)
        c1 = pltpu.make_async_copy(xbuf, xs_hbm.at[d1], sem)
        c0.start()
        c1.start()
        c0.wait()
        c1.wait()
        return carry

    lax.fori_loop(0, NCH, chunk, 0)


def _combine_body(y_hbm, p0_hbm, p1_hbm, v0_hbm, v1_hbm, out_hbm,
                  b0a, b1a, b0b, b1b, oba, obb, d0, d1, w0, w1,
                  sem_ga, sem_gb, sem_sa, sem_sb):
    wid = lax.axis_index("s") * SC_CORES + lax.axis_index("c")
    base = wid * TOK_W
    pltpu.sync_copy(p0_hbm.at[pl.ds(base, TOK_W)], d0)
    pltpu.sync_copy(p1_hbm.at[pl.ds(base, TOK_W)], d1)
    pltpu.sync_copy(v0_hbm.at[pl.ds(base, TOK_W)], w0)
    pltpu.sync_copy(v1_hbm.at[pl.ds(base, TOK_W)], w1)

    bufs = ((b0a, b1a, oba, sem_ga, sem_sa), (b0b, b1b, obb, sem_gb, sem_sb))

    def fire(k, bi):
        b0, b1, _, sem_g, _ = bufs[bi]
        sl = pl.ds(k * CCH, CCH)
        pltpu.make_async_copy(y_hbm.at[d0.at[sl]], b0, sem_g).start()
        pltpu.make_async_copy(y_hbm.at[d1.at[sl]], b1, sem_g).start()

    gdn = lax.GatherDimensionNumbers(offset_dims=(),
                                     collapsed_slice_dims=(0,),
                                     start_index_map=(0,))
    UN = 8

    def consume(k, bi):
        b0, b1, ob, sem_g, sem_s = bufs[bi]
        pltpu.make_async_copy(y_hbm.at[d0.at[pl.ds(0, CCH)]], b0, sem_g).wait()
        pltpu.make_async_copy(y_hbm.at[d1.at[pl.ds(0, CCH)]], b1, sem_g).wait()

        def row(r, rcarry):
            ridx = jnp.zeros((LANES, 1), jnp.int32) + (k * CCH + r)
            wv0 = lax.gather(w0[...], ridx, gdn, (1,),
                             mode=lax.GatherScatterMode.PROMISE_IN_BOUNDS) * 0.5
            wv1 = lax.gather(w1[...], ridx, gdn, (1,),
                             mode=lax.GatherScatterMode.PROMISE_IN_BOUNDS) * 0.5

            def col(c, ccarry):
                for u in range(UN):
                    sl = pl.ds((c * UN + u) * LANES, LANES)
                    ob[r, sl] = b0[r, sl] * wv0 + b1[r, sl] * wv1
                return ccarry

            lax.fori_loop(0, NUM_CLASSES // (LANES * UN), col, 0)
            return rcarry

        lax.fori_loop(0, CCH, row, 0)
        tb = pl.multiple_of(base + k * CCH, CCH)
        pltpu.make_async_copy(ob, out_hbm.at[pl.ds(tb, CCH)], sem_s).start()

    def drain_store(bi):
        _, _, ob, _, sem_s = bufs[bi]
        pltpu.make_async_copy(ob, out_hbm.at[pl.ds(base, CCH)], sem_s).wait()

    fire(0, 0)
    for k in range(NCH):
        bi = k % 2
        if k + 1 < NCH:
            fire(k + 1, 1 - bi)
        if k >= 2:
            drain_store(bi)
        consume(k, bi)
    drain_store(0 if NCH % 2 == 0 else 1)
    drain_store(1 if NCH % 2 == 0 else 0)


_SC_MESH = plsc.VectorSubcoreMesh(core_axis_name="c", subcore_axis_name="s")


def _sc_dispatch(x, pos0, pos1):
    return pl.kernel(
        _dispatch_body,
        out_type=jax.ShapeDtypeStruct((PPAD, IN_DIM), jnp.float32),
        mesh=_SC_MESH,
        scratch_types=[
            pltpu.VMEM((CCH, IN_DIM), jnp.float32),
            pltpu.VMEM((CCH,), jnp.int32),
            pltpu.VMEM((CCH,), jnp.int32),
            pltpu.SemaphoreType.DMA,
        ],
    )(x, pos0, pos1)


def _sc_combine(y, pos0, pos1, v0, v1):
    return pl.kernel(
        _combine_body,
        out_type=jax.ShapeDtypeStruct((B, NUM_CLASSES), jnp.float32),
        mesh=_SC_MESH,
        scratch_types=[
            pltpu.VMEM((CCH, NUM_CLASSES), jnp.float32),
            pltpu.VMEM((CCH, NUM_CLASSES), jnp.float32),
            pltpu.VMEM((CCH, NUM_CLASSES), jnp.float32),
            pltpu.VMEM((CCH, NUM_CLASSES), jnp.float32),
            pltpu.VMEM((CCH, NUM_CLASSES), jnp.float32),
            pltpu.VMEM((CCH, NUM_CLASSES), jnp.float32),
            pltpu.VMEM((TOK_W,), jnp.int32),
            pltpu.VMEM((TOK_W,), jnp.int32),
            pltpu.VMEM((TOK_W,), jnp.float32),
            pltpu.VMEM((TOK_W,), jnp.float32),
            pltpu.SemaphoreType.DMA,
            pltpu.SemaphoreType.DMA,
            pltpu.SemaphoreType.DMA,
            pltpu.SemaphoreType.DMA,
        ],
    )(y, pos0, pos1, v0, v1)


def kernel(x, Wr1, br1, Wr2, br2, We1, be1, We2, be2):
    probs, pos0, pos1, v0, v1, tile_e, loss = _router(x, Wr1, br1, Wr2, br2)
    pos0 = pos0.reshape(B)
    pos1 = pos1.reshape(B)
    tile_e = tile_e.reshape(NMT)

    xs = _sc_dispatch(x, pos0, pos1)

    h = _grouped_gemm(xs, We1, be1, tile_e, _gemm1_body, EXPERT_HIDDEN,
                      jnp.bfloat16)
    y = _grouped_gemm(h, We2, be2, tile_e, _gemm2_body, NUM_CLASSES,
                      jnp.float32)

    out = _sc_combine(y, pos0, pos1, v0.reshape(B), v1.reshape(B))
    return out, probs, loss[0, 0]


# speedup vs baseline: 2.0781x; 1.0013x over previous
import functools
from typing import Any

import jax
import jax.numpy as jnp
from jax import lax
from jax.experimental import pallas as pl
from jax.experimental.pallas import tpu as pltpu
from jax.experimental.pallas import tpu_sc as plsc

B = 4096
IN_DIM = 1024
NUM_CLASSES = 1024
NUM_EXPERTS = 8
TOP_K = 2
ROUTER_HIDDEN = 256
EXPERT_HIDDEN = 4096

TM = 256
PPAD = B * TOP_K + NUM_EXPERTS * TM
NMT = PPAD // TM
CHUNK = 512

SC_CORES = 2
SC_SUBCORES = 16
NW = SC_CORES * SC_SUBCORES
TOK_W = B // NW
CCH = 16
NCH = TOK_W // CCH
LANES = 16


def _router_body(x_ref, wr1_ref, br1_ref, wr2_ref, br2_ref,
                 probs_ref, pos0_ref, pos1_ref, v0_ref, v1_ref,
                 tile_e_ref, loss_ref):
    x = x_ref[...]
    h = jnp.maximum(jnp.dot(x, wr1_ref[...],
                            preferred_element_type=jnp.float32)
                    + br1_ref[...], 0.0)
    s = jnp.dot(h, wr2_ref[...], preferred_element_type=jnp.float32) \
        + br2_ref[...]
    m = jnp.max(s, axis=1, keepdims=True)
    p = jnp.exp(s - m)
    p = p / jnp.sum(p, axis=1, keepdims=True)
    probs_ref[...] = p

    loss_vec = jnp.sum(p, axis=0, keepdims=True) * (1.0 / B)
    loss_ref[...] = jnp.sum(loss_vec * loss_vec).reshape(1, 1)

    eidx = jax.lax.broadcasted_iota(jnp.int32, (B, NUM_EXPERTS), 1)
    m0 = jnp.max(p, axis=1, keepdims=True)
    e0 = jnp.min(jnp.where(p == m0, eidx, NUM_EXPERTS), axis=1, keepdims=True)
    oh0 = (eidx == e0).astype(jnp.float32)
    pm = jnp.where(oh0 > 0, -jnp.inf, p)
    m1 = jnp.max(pm, axis=1, keepdims=True)
    e1 = jnp.min(jnp.where(pm == m1, eidx, NUM_EXPERTS), axis=1, keepdims=True)
    oh1 = (eidx == e1).astype(jnp.float32)
    v0_ref[...] = m0
    v1_ref[...] = m1

    mm = oh0 + oh1
    cnt = jnp.sum(mm, axis=0, keepdims=True)
    cnti = cnt.astype(jnp.int32)
    padded = ((cnti + (TM - 1)) // TM) * TM
    ltri8 = (jax.lax.broadcasted_iota(jnp.int32, (NUM_EXPERTS, NUM_EXPERTS), 0)
             < jax.lax.broadcasted_iota(jnp.int32, (NUM_EXPERTS, NUM_EXPERTS), 1)
             ).astype(jnp.float32)
    off = jnp.dot(padded.astype(jnp.float32), ltri8,
                  preferred_element_type=jnp.float32)

    tstart = jax.lax.broadcasted_iota(jnp.int32, (NMT, NUM_EXPERTS), 0) * TM
    grp_end = (off + padded.astype(jnp.float32)).astype(jnp.int32)
    tile_e = jnp.sum((tstart >= grp_end).astype(jnp.int32), axis=1,
                     keepdims=True)
    tile_e_ref[...] = jnp.minimum(tile_e, NUM_EXPERTS - 1)

    ltri = (jax.lax.broadcasted_iota(jnp.int32, (CHUNK, CHUNK), 1)
            < jax.lax.broadcasted_iota(jnp.int32, (CHUNK, CHUNK), 0)
            ).astype(jnp.float32)
    run = jnp.zeros((1, NUM_EXPERTS), jnp.float32)
    for c in range(B // CHUNK):
        sl = slice(c * CHUNK, (c + 1) * CHUNK)
        mm_c = mm[sl]
        t = jnp.dot(ltri, mm_c, preferred_element_type=jnp.float32)
        base = off + run + t
        p0 = jnp.sum(oh0[sl] * base, axis=1, keepdims=True)
        p1 = jnp.sum(oh1[sl] * (base + oh0[sl]), axis=1, keepdims=True)
        pos0_ref[sl, :] = p0.astype(jnp.int32)
        pos1_ref[sl, :] = p1.astype(jnp.int32)
        run = run + jnp.sum(mm_c, axis=0, keepdims=True)


@functools.partial(jax.jit, static_argnames=("interpret",))
def _router(x, wr1, br1, wr2, br2, interpret=False):
    outs = pl.pallas_call(
        _router_body,
        out_shape=(
            jax.ShapeDtypeStruct((B, NUM_EXPERTS), jnp.float32),
            jax.ShapeDtypeStruct((B, 1), jnp.int32),
            jax.ShapeDtypeStruct((B, 1), jnp.int32),
            jax.ShapeDtypeStruct((B, 1), jnp.float32),
            jax.ShapeDtypeStruct((B, 1), jnp.float32),
            jax.ShapeDtypeStruct((NMT, 1), jnp.int32),
            jax.ShapeDtypeStruct((1, 1), jnp.float32),
        ),
        interpret=interpret,
    )(x, wr1, br1.reshape(1, ROUTER_HIDDEN), wr2, br2.reshape(1, NUM_EXPERTS))
    return outs


def _gemm1_body(tile_e_ref, xs_ref, w_ref, b_ref, out_ref):
    acc = jnp.dot(xs_ref[...].astype(jnp.bfloat16),
                  w_ref[0].astype(jnp.bfloat16),
                  preferred_element_type=jnp.float32)
    out_ref[...] = jnp.maximum(acc + b_ref[0], 0.0).astype(jnp.bfloat16)


def _gemm2_body(tile_e_ref, h_ref, w_ref, b_ref, out_ref):
    acc = jnp.dot(h_ref[...], w_ref[0].astype(jnp.bfloat16),
                  preferred_element_type=jnp.float32)
    out_ref[...] = acc + b_ref[0]


def _grouped_gemm(xs, w, b, tile_e, body, tn, out_dtype, interpret=False):
    k = xs.shape[1]
    n = w.shape[2]
    grid = (n // tn, NMT)
    return pl.pallas_call(
        body,
        grid_spec=pltpu.PrefetchScalarGridSpec(
            num_scalar_prefetch=1,
            grid=grid,
            in_specs=[
                pl.BlockSpec((TM, k), lambda ni, mi, te: (mi, 0)),
                pl.BlockSpec((1, k, tn), lambda ni, mi, te: (te[mi], 0, ni)),
                pl.BlockSpec((1, 1, tn), lambda ni, mi, te: (te[mi], 0, ni)),
            ],
            out_specs=pl.BlockSpec((TM, tn), lambda ni, mi, te: (mi, ni)),
        ),
        out_shape=jax.ShapeDtypeStruct((PPAD, n), out_dtype),
        interpret=interpret,
    )(tile_e, xs, w, b.reshape(NUM_EXPERTS, 1, n))


def _dispatch_body(x_hbm, p0_hbm, p1_hbm, xs_hbm, xbuf, d0, d1, sem):
    wid = lax.axis_index("s") * SC_CORES + lax.axis_index("c")
    base = wid * TOK_W

    def chunk(k, carry):
        tb = pl.multiple_of(base + k * CCH, CCH)
        pltpu.sync_copy(p0_hbm.at[pl.ds(tb, CCH)], d0)
        pltpu.sync_copy(p1_hbm.at[pl.ds(tb, CCH)], d1)
        pltpu.sync_copy(x_hbm.at[pl.ds(tb, CCH)], xbuf)
        c0 = pltpu.make_async_copy(xbuf, xs_hbm.at[d0], sem)
        c1 = pltpu.make_async_copy(xbuf, xs_hbm.at[d1], sem)
        c0.start()
        c1.start()
        c0.wait()
        c1.wait()
        return carry

    lax.fori_loop(0, NCH, chunk, 0)


def _combine_body(y_hbm, p0_hbm, p1_hbm, v0_hbm, v1_hbm, out_hbm,
                  b01a, b01b, oba, obb, d0, d1, d01, w0, w1,
                  sem_ga, sem_gb, sem_sa, sem_sb):
    wid = lax.axis_index("s") * SC_CORES + lax.axis_index("c")
    base = wid * TOK_W
    pltpu.sync_copy(p0_hbm.at[pl.ds(base, TOK_W)], d0)
    pltpu.sync_copy(p1_hbm.at[pl.ds(base, TOK_W)], d1)
    pltpu.sync_copy(v0_hbm.at[pl.ds(base, TOK_W)], w0)
    pltpu.sync_copy(v1_hbm.at[pl.ds(base, TOK_W)], w1)

    for k in range(NCH):
        sl = pl.ds(k * CCH, CCH)
        d01[pl.ds(2 * k * CCH, CCH)] = d0[sl]
        d01[pl.ds((2 * k + 1) * CCH, CCH)] = d1[sl]

    bufs = ((b01a, oba, sem_ga, sem_sa), (b01b, obb, sem_gb, sem_sb))

    def fire(k, bi):
        b01, _, sem_g, _ = bufs[bi]
        pltpu.make_async_copy(y_hbm.at[d01.at[pl.ds(2 * k * CCH, 2 * CCH)]],
                              b01, sem_g).start()

    gdn = lax.GatherDimensionNumbers(offset_dims=(),
                                     collapsed_slice_dims=(0,),
                                     start_index_map=(0,))
    UN = 8

    def consume(k, bi):
        b01, ob, sem_g, sem_s = bufs[bi]
        pltpu.make_async_copy(y_hbm.at[d01.at[pl.ds(0, 2 * CCH)]],
                              b01, sem_g).wait()

        def row(r, rcarry):
            ridx = jnp.zeros((LANES, 1), jnp.int32) + (k * CCH + r)
            wv0 = lax.gather(w0[...], ridx, gdn, (1,),
                             mode=lax.GatherScatterMode.PROMISE_IN_BOUNDS) * 0.5
            wv1 = lax.gather(w1[...], ridx, gdn, (1,),
                             mode=lax.GatherScatterMode.PROMISE_IN_BOUNDS) * 0.5

            def col(c, ccarry):
                for u in range(UN):
                    sl = pl.ds((c * UN + u) * LANES, LANES)
                    ob[r, sl] = b01[r, sl] * wv0 + b01[CCH + r, sl] * wv1
                return ccarry

            lax.fori_loop(0, NUM_CLASSES // (LANES * UN), col, 0)
            return rcarry

        lax.fori_loop(0, CCH, row, 0)
        tb = pl.multiple_of(base + k * CCH, CCH)
        pltpu.make_async_copy(ob, out_hbm.at[pl.ds(tb, CCH)], sem_s).start()

    def drain_store(bi):
        _, ob, _, sem_s = bufs[bi]
        pltpu.make_async_copy(ob, out_hbm.at[pl.ds(base, CCH)], sem_s).wait()

    fire(0, 0)
    for k in range(NCH):
        bi = k % 2
        if k + 1 < NCH:
            fire(k + 1, 1 - bi)
        if k >= 2:
            drain_store(bi)
        consume(k, bi)
    drain_store(0 if NCH % 2 == 0 else 1)
    drain_store(1 if NCH % 2 == 0 else 0)


_SC_MESH = plsc.VectorSubcoreMesh(core_axis_name="c", subcore_axis_name="s")


def _sc_dispatch(x, pos0, pos1):
    return pl.kernel(
        _dispatch_body,
        out_type=jax.ShapeDtypeStruct((PPAD, IN_DIM), jnp.float32),
        mesh=_SC_MESH,
        scratch_types=[
            pltpu.VMEM((CCH, IN_DIM), jnp.float32),
            pltpu.VMEM((CCH,), jnp.int32),
            pltpu.VMEM((CCH,), jnp.int32),
            pltpu.SemaphoreType.DMA,
        ],
    )(x, pos0, pos1)


def _sc_combine(y, pos0, pos1, v0, v1):
    return pl.kernel(
        _combine_body,
        out_type=jax.ShapeDtypeStruct((B, NUM_CLASSES), jnp.float32),
        mesh=_SC_MESH,
        scratch_types=[
            pltpu.VMEM((2 * CCH, NUM_CLASSES), jnp.float32),
            pltpu.VMEM((2 * CCH, NUM_CLASSES), jnp.float32),
            pltpu.VMEM((CCH, NUM_CLASSES), jnp.float32),
            pltpu.VMEM((CCH, NUM_CLASSES), jnp.float32),
            pltpu.VMEM((TOK_W,), jnp.int32),
            pltpu.VMEM((TOK_W,), jnp.int32),
            pltpu.VMEM((2 * TOK_W,), jnp.int32),
            pltpu.VMEM((TOK_W,), jnp.float32),
            pltpu.VMEM((TOK_W,), jnp.float32),
            pltpu.SemaphoreType.DMA,
            pltpu.SemaphoreType.DMA,
            pltpu.SemaphoreType.DMA,
            pltpu.SemaphoreType.DMA,
        ],
    )(y, pos0, pos1, v0, v1)


def kernel(x, Wr1, br1, Wr2, br2, We1, be1, We2, be2):
    probs, pos0, pos1, v0, v1, tile_e, loss = _router(x, Wr1, br1, Wr2, br2)
    pos0 = pos0.reshape(B)
    pos1 = pos1.reshape(B)
    tile_e = tile_e.reshape(NMT)

    xs = _sc_dispatch(x, pos0, pos1)

    h = _grouped_gemm(xs, We1, be1, tile_e, _gemm1_body, EXPERT_HIDDEN,
                      jnp.bfloat16)
    y = _grouped_gemm(h, We2, be2, tile_e, _gemm2_body, NUM_CLASSES,
                      jnp.float32)

    out = _sc_combine(y, pos0, pos1, v0.reshape(B), v1.reshape(B))
    return out, probs, loss[0, 0]


# speedup vs baseline: 2.2536x; 1.0844x over previous
import functools
from typing import Any

import jax
import jax.numpy as jnp
from jax import lax
from jax.experimental import pallas as pl
from jax.experimental.pallas import tpu as pltpu
from jax.experimental.pallas import tpu_sc as plsc

B = 4096
IN_DIM = 1024
NUM_CLASSES = 1024
NUM_EXPERTS = 8
TOP_K = 2
ROUTER_HIDDEN = 256
EXPERT_HIDDEN = 4096

TM = 256
PPAD = B * TOP_K + NUM_EXPERTS * TM
NMT = PPAD // TM
CHUNK = 512

SC_CORES = 2
SC_SUBCORES = 16
NW = SC_CORES * SC_SUBCORES
TOK_W = B // NW
CCH = 16
NCH = TOK_W // CCH
LANES = 16


def _router_body(x_ref, wr1_ref, br1_ref, wr2_ref, br2_ref,
                 probs_ref, pos0_ref, pos1_ref, v0_ref, v1_ref,
                 tile_e_ref, loss_ref):
    x = x_ref[...]
    h = jnp.maximum(jnp.dot(x, wr1_ref[...],
                            preferred_element_type=jnp.float32)
                    + br1_ref[...], 0.0)
    s = jnp.dot(h, wr2_ref[...], preferred_element_type=jnp.float32) \
        + br2_ref[...]
    m = jnp.max(s, axis=1, keepdims=True)
    p = jnp.exp(s - m)
    p = p / jnp.sum(p, axis=1, keepdims=True)
    probs_ref[...] = p

    loss_vec = jnp.sum(p, axis=0, keepdims=True) * (1.0 / B)
    loss_ref[...] = jnp.sum(loss_vec * loss_vec).reshape(1, 1)

    eidx = jax.lax.broadcasted_iota(jnp.int32, (B, NUM_EXPERTS), 1)
    m0 = jnp.max(p, axis=1, keepdims=True)
    e0 = jnp.min(jnp.where(p == m0, eidx, NUM_EXPERTS), axis=1, keepdims=True)
    oh0 = (eidx == e0).astype(jnp.float32)
    pm = jnp.where(oh0 > 0, -jnp.inf, p)
    m1 = jnp.max(pm, axis=1, keepdims=True)
    e1 = jnp.min(jnp.where(pm == m1, eidx, NUM_EXPERTS), axis=1, keepdims=True)
    oh1 = (eidx == e1).astype(jnp.float32)
    v0_ref[...] = m0
    v1_ref[...] = m1

    mm = oh0 + oh1
    cnt = jnp.sum(mm, axis=0, keepdims=True)
    cnti = cnt.astype(jnp.int32)
    padded = ((cnti + (TM - 1)) // TM) * TM
    ltri8 = (jax.lax.broadcasted_iota(jnp.int32, (NUM_EXPERTS, NUM_EXPERTS), 0)
             < jax.lax.broadcasted_iota(jnp.int32, (NUM_EXPERTS, NUM_EXPERTS), 1)
             ).astype(jnp.float32)
    off = jnp.dot(padded.astype(jnp.float32), ltri8,
                  preferred_element_type=jnp.float32)

    tstart = jax.lax.broadcasted_iota(jnp.int32, (NMT, NUM_EXPERTS), 0) * TM
    grp_end = (off + padded.astype(jnp.float32)).astype(jnp.int32)
    tile_e = jnp.sum((tstart >= grp_end).astype(jnp.int32), axis=1,
                     keepdims=True)
    tile_e_ref[...] = jnp.minimum(tile_e, NUM_EXPERTS - 1)

    ltri = (jax.lax.broadcasted_iota(jnp.int32, (CHUNK, CHUNK), 1)
            < jax.lax.broadcasted_iota(jnp.int32, (CHUNK, CHUNK), 0)
            ).astype(jnp.float32)
    run = jnp.zeros((1, NUM_EXPERTS), jnp.float32)
    for c in range(B // CHUNK):
        sl = slice(c * CHUNK, (c + 1) * CHUNK)
        mm_c = mm[sl]
        t = jnp.dot(ltri, mm_c, preferred_element_type=jnp.float32)
        base = off + run + t
        p0 = jnp.sum(oh0[sl] * base, axis=1, keepdims=True)
        p1 = jnp.sum(oh1[sl] * (base + oh0[sl]), axis=1, keepdims=True)
        pos0_ref[sl, :] = p0.astype(jnp.int32)
        pos1_ref[sl, :] = p1.astype(jnp.int32)
        run = run + jnp.sum(mm_c, axis=0, keepdims=True)


@functools.partial(jax.jit, static_argnames=("interpret",))
def _router(x, wr1, br1, wr2, br2, interpret=False):
    outs = pl.pallas_call(
        _router_body,
        out_shape=(
            jax.ShapeDtypeStruct((B, NUM_EXPERTS), jnp.float32),
            jax.ShapeDtypeStruct((B, 1), jnp.int32),
            jax.ShapeDtypeStruct((B, 1), jnp.int32),
            jax.ShapeDtypeStruct((B, 1), jnp.float32),
            jax.ShapeDtypeStruct((B, 1), jnp.float32),
            jax.ShapeDtypeStruct((NMT, 1), jnp.int32),
            jax.ShapeDtypeStruct((1, 1), jnp.float32),
        ),
        interpret=interpret,
    )(x, wr1, br1.reshape(1, ROUTER_HIDDEN), wr2, br2.reshape(1, NUM_EXPERTS))
    return outs


def _gemm1_body(tile_e_ref, xs_ref, w_ref, b_ref, out_ref):
    acc = jnp.dot(xs_ref[...].astype(jnp.bfloat16),
                  w_ref[0].astype(jnp.bfloat16),
                  preferred_element_type=jnp.float32)
    out_ref[...] = jnp.maximum(acc + b_ref[0], 0.0).astype(jnp.bfloat16)


def _gemm2_body(tile_e_ref, h_ref, w_ref, b_ref, out_ref):
    acc = jnp.dot(h_ref[...], w_ref[0].astype(jnp.bfloat16),
                  preferred_element_type=jnp.float32)
    out_ref[...] = acc + b_ref[0]


def _grouped_gemm(xs, w, b, tile_e, body, tn, out_dtype, interpret=False):
    k = xs.shape[1]
    n = w.shape[2]
    grid = (n // tn, NMT)
    return pl.pallas_call(
        body,
        grid_spec=pltpu.PrefetchScalarGridSpec(
            num_scalar_prefetch=1,
            grid=grid,
            in_specs=[
                pl.BlockSpec((TM, k), lambda ni, mi, te: (mi, 0)),
                pl.BlockSpec((1, k, tn), lambda ni, mi, te: (te[mi], 0, ni)),
                pl.BlockSpec((1, 1, tn), lambda ni, mi, te: (te[mi], 0, ni)),
            ],
            out_specs=pl.BlockSpec((TM, tn), lambda ni, mi, te: (mi, ni)),
        ),
        out_shape=jax.ShapeDtypeStruct((PPAD, n), out_dtype),
        interpret=interpret,
    )(tile_e, xs, w, b.reshape(NUM_EXPERTS, 1, n))


def _dispatch_body(x_hbm, p0_hbm, p1_hbm, xs_hbm, xbuf, d0, d1, sem):
    wid = lax.axis_index("s") * SC_CORES + lax.axis_index("c")
    base = wid * TOK_W

    def chunk(k, carry):
        tb = pl.multiple_of(base + k * CCH, CCH)
        pltpu.sync_copy(p0_hbm.at[pl.ds(tb, CCH)], d0)
        pltpu.sync_copy(p1_hbm.at[pl.ds(tb, CCH)], d1)
        pltpu.sync_copy(x_hbm.at[pl.ds(tb, CCH)], xbuf)
        c0 = pltpu.make_async_copy(xbuf, xs_hbm.at[d0], sem)
        c1 = pltpu.make_async_copy(xbuf, xs_hbm.at[d1], sem)
        c0.start()
        c1.start()
        c0.wait()
        c1.wait()
        return carry

    lax.fori_loop(0, NCH, chunk, 0)


def _combine_body(y_hbm, p0_hbm, p1_hbm, v0_hbm, v1_hbm, out_hbm,
                  b01a, b01b, oba, obb, d0, d1, d01, w0, w1,
                  sem_ga, sem_gb, sem_sa, sem_sb):
    wid = lax.axis_index("s") * SC_CORES + lax.axis_index("c")
    base = wid * TOK_W
    pltpu.sync_copy(p0_hbm.at[pl.ds(base, TOK_W)], d0)
    pltpu.sync_copy(p1_hbm.at[pl.ds(base, TOK_W)], d1)
    pltpu.sync_copy(v0_hbm.at[pl.ds(base, TOK_W)], w0)
    pltpu.sync_copy(v1_hbm.at[pl.ds(base, TOK_W)], w1)

    for k in range(NCH):
        sl = pl.ds(k * CCH, CCH)
        d01[pl.ds(2 * k * CCH, CCH)] = d0[sl]
        d01[pl.ds((2 * k + 1) * CCH, CCH)] = d1[sl]

    bufs = ((b01a, oba, sem_ga, sem_sa), (b01b, obb, sem_gb, sem_sb))

    def fire(k, bi):
        b01, _, sem_g, _ = bufs[bi]
        pltpu.make_async_copy(y_hbm.at[d01.at[pl.ds(2 * k * CCH, 2 * CCH)]],
                              b01, sem_g).start()

    gdn = lax.GatherDimensionNumbers(offset_dims=(),
                                     collapsed_slice_dims=(0,),
                                     start_index_map=(0,))
    UN = 8

    def consume(k, bi, wait_store):
        b01, ob, sem_g, sem_s = bufs[bi]
        pltpu.make_async_copy(y_hbm.at[d01.at[pl.ds(0, 2 * CCH)]],
                              b01, sem_g).wait()
        if wait_store:
            pltpu.make_async_copy(ob, out_hbm.at[pl.ds(base, CCH)],
                                  sem_s).wait()
        for r in range(CCH):
            ridx = jnp.zeros((LANES, 1), jnp.int32) + (k * CCH + r)
            wv0 = lax.gather(w0[...], ridx, gdn, (1,),
                             mode=lax.GatherScatterMode.PROMISE_IN_BOUNDS) * 0.5
            wv1 = lax.gather(w1[...], ridx, gdn, (1,),
                             mode=lax.GatherScatterMode.PROMISE_IN_BOUNDS) * 0.5

            def col(c, ccarry, _r=r, _wv0=wv0, _wv1=wv1):
                for u in range(UN):
                    sl = pl.ds((c * UN + u) * LANES, LANES)
                    ob[_r, sl] = (b01[_r, sl] * _wv0
                                  + b01[CCH + _r, sl] * _wv1)
                return ccarry

            lax.fori_loop(0, NUM_CLASSES // (LANES * UN), col, 0)
        tb = pl.multiple_of(base + k * CCH, CCH)
        pltpu.make_async_copy(ob, out_hbm.at[pl.ds(tb, CCH)], sem_s).start()

    def drain_store(bi):
        _, ob, _, sem_s = bufs[bi]
        pltpu.make_async_copy(ob, out_hbm.at[pl.ds(base, CCH)], sem_s).wait()

    fire(0, 0)
    fire(1, 1)
    consume(0, 0, wait_store=False)
    fire(2, 0)
    consume(1, 1, wait_store=False)

    def pair2(j, carry):
        k0 = 2 + j * 2
        fire(k0 + 1, 1)
        consume(k0, 0, wait_store=True)

        @pl.when(j + 2 < NCH // 2)
        def _():
            fire(k0 + 2, 0)

        consume(k0 + 1, 1, wait_store=True)
        return carry

    lax.fori_loop(0, NCH // 2 - 1, pair2, 0)
    drain_store(0)
    drain_store(1)


_SC_MESH = plsc.VectorSubcoreMesh(core_axis_name="c", subcore_axis_name="s")


def _sc_dispatch(x, pos0, pos1):
    return pl.kernel(
        _dispatch_body,
        out_type=jax.ShapeDtypeStruct((PPAD, IN_DIM), jnp.float32),
        mesh=_SC_MESH,
        scratch_types=[
            pltpu.VMEM((CCH, IN_DIM), jnp.float32),
            pltpu.VMEM((CCH,), jnp.int32),
            pltpu.VMEM((CCH,), jnp.int32),
            pltpu.SemaphoreType.DMA,
        ],
    )(x, pos0, pos1)


def _sc_combine(y, pos0, pos1, v0, v1):
    return pl.kernel(
        _combine_body,
        out_type=jax.ShapeDtypeStruct((B, NUM_CLASSES), jnp.float32),
        mesh=_SC_MESH,
        scratch_types=[
            pltpu.VMEM((2 * CCH, NUM_CLASSES), jnp.float32),
            pltpu.VMEM((2 * CCH, NUM_CLASSES), jnp.float32),
            pltpu.VMEM((CCH, NUM_CLASSES), jnp.float32),
            pltpu.VMEM((CCH, NUM_CLASSES), jnp.float32),
            pltpu.VMEM((TOK_W,), jnp.int32),
            pltpu.VMEM((TOK_W,), jnp.int32),
            pltpu.VMEM((2 * TOK_W,), jnp.int32),
            pltpu.VMEM((TOK_W,), jnp.float32),
            pltpu.VMEM((TOK_W,), jnp.float32),
            pltpu.SemaphoreType.DMA,
            pltpu.SemaphoreType.DMA,
            pltpu.SemaphoreType.DMA,
            pltpu.SemaphoreType.DMA,
        ],
    )(y, pos0, pos1, v0, v1)


def kernel(x, Wr1, br1, Wr2, br2, We1, be1, We2, be2):
    probs, pos0, pos1, v0, v1, tile_e, loss = _router(x, Wr1, br1, Wr2, br2)
    pos0 = pos0.reshape(B)
    pos1 = pos1.reshape(B)
    tile_e = tile_e.reshape(NMT)

    xs = _sc_dispatch(x, pos0, pos1)

    h = _grouped_gemm(xs, We1, be1, tile_e, _gemm1_body, EXPERT_HIDDEN,
                      jnp.bfloat16)
    y = _grouped_gemm(h, We2, be2, tile_e, _gemm2_body, NUM_CLASSES,
                      jnp.float32)

    out = _sc_combine(y, pos0, pos1, v0.reshape(B), v1.reshape(B))
    return out, probs, loss[0, 0]
